# baseline (speedup 1.0000x reference)
import functools

import jax
import jax.numpy as jnp
from jax import lax
from jax.experimental import pallas as pl
from jax.experimental.pallas import tpu as pltpu
from jax.experimental.pallas import tpu_sc as plsc

F_DIM = 26
E_DIM = 16
NC = 2
NS = 16
NW = NC * NS
CHUNK = 128
TBLK = 2048


def _tr_body(tin_ref, tout_ref):
    tout_ref[...] = tin_ref[...].T


def _tc_transpose(table_t):
    _, V = table_t.shape
    grid = (pl.cdiv(V, TBLK),)
    return pl.pallas_call(
        _tr_body,
        grid=grid,
        in_specs=[pl.BlockSpec((E_DIM, TBLK), lambda i: (0, i))],
        out_specs=pl.BlockSpec((TBLK, E_DIM), lambda i: (i, 0)),
        out_shape=jax.ShapeDtypeStruct((V, E_DIM), jnp.float32),
    )(table_t)


_SC_MESH = plsc.VectorSubcoreMesh(core_axis_name="c", subcore_axis_name="s")


def _sc_gather_emb(idx_r, table):
    C = idx_r.shape[1]
    n_per_w = C * CHUNK

    @functools.partial(
        pl.kernel,
        out_type=jax.ShapeDtypeStruct((NW, n_per_w, E_DIM), jnp.float32),
        mesh=_SC_MESH,
        scratch_types=[
            pltpu.VMEM((C, CHUNK), jnp.int32),
            pltpu.VMEM((n_per_w, E_DIM), jnp.float32),
            pltpu.SemaphoreType.DMA,
        ],
        compiler_params=pltpu.CompilerParams(use_tc_tiling_on_sc=False),
    )
    def sc_kernel(idx_hbm, table_hbm, emb_out, idx_v, rows_v, sem):
        wid = lax.axis_index("s") * NC + lax.axis_index("c")
        pltpu.sync_copy(idx_hbm.at[wid], idx_v)

        def fire(ci, _):
            pltpu.async_copy(
                table_hbm.at[idx_v.at[ci]],
                rows_v.at[pl.ds(ci * CHUNK, CHUNK)], sem)
            return 0

        lax.fori_loop(0, C, fire, 0)

        def drain(ci, _):
            pltpu.make_async_copy(
                table_hbm.at[idx_v.at[ci]],
                rows_v.at[pl.ds(ci * CHUNK, CHUNK)], sem).wait()
            return 0

        lax.fori_loop(0, C, drain, 0)
        pltpu.sync_copy(rows_v, emb_out.at[wid])

    return sc_kernel(idx_r, table)


def _sc_gather_lin(idx_r, lin_w):
    C = idx_r.shape[1]
    n_per_w = C * CHUNK

    @functools.partial(
        pl.kernel,
        out_type=jax.ShapeDtypeStruct((NW, n_per_w), jnp.float32),
        mesh=_SC_MESH,
        scratch_types=[
            pltpu.VMEM((C, CHUNK), jnp.int32),
            pltpu.VMEM((n_per_w,), jnp.float32),
            pltpu.SemaphoreType.DMA,
        ],
        compiler_params=pltpu.CompilerParams(use_tc_tiling_on_sc=False),
    )
    def sc_kernel(idx_hbm, lin_hbm, lin_out, idx_v, linr_v, sem):
        wid = lax.axis_index("s") * NC + lax.axis_index("c")
        pltpu.sync_copy(idx_hbm.at[wid], idx_v)

        def fire(ci, _):
            pltpu.async_copy(
                lin_hbm.at[idx_v.at[ci]],
                linr_v.at[pl.ds(ci * CHUNK, CHUNK)], sem)
            return 0

        lax.fori_loop(0, C, fire, 0)

        def drain(ci, _):
            pltpu.make_async_copy(
                lin_hbm.at[idx_v.at[ci]],
                linr_v.at[pl.ds(ci * CHUNK, CHUNK)], sem).wait()
            return 0

        lax.fori_loop(0, C, drain, 0)
        pltpu.sync_copy(linr_v, lin_out.at[wid])

    return sc_kernel(idx_r, lin_w)


def _tc_body(emb_ref, vals_ref, ling_ref, W1_ref, b1_ref, W2_ref, b2_ref,
             Wp_ref, bp_ref, lb_ref, out_ref):
    emb = emb_ref[...]
    vals = vals_ref[...]
    ling = ling_ref[...]

    fe = F_DIM * E_DIM
    jf = lax.broadcasted_iota(jnp.int32, (F_DIM, fe), 1) // E_DIM
    ff = lax.broadcasted_iota(jnp.int32, (F_DIM, fe), 0)
    expand = (jf == ff).astype(jnp.float32)
    w = emb * jnp.dot(vals, expand, preferred_element_type=jnp.float32)

    linear = jnp.sum(ling * vals, axis=1, keepdims=True) + lb_ref[0, 0]

    jj = lax.broadcasted_iota(jnp.int32, (fe, E_DIM), 0)
    dd = lax.broadcasted_iota(jnp.int32, (fe, E_DIM), 1)
    fold = (jj % E_DIM == dd).astype(jnp.float32)
    s = jnp.dot(w, fold, preferred_element_type=jnp.float32)
    fm = 0.5 * (jnp.sum(s * s, axis=1, keepdims=True)
                - jnp.sum(w * w, axis=1, keepdims=True))

    h = jnp.maximum(
        jnp.dot(w, W1_ref[...], preferred_element_type=jnp.float32)
        + b1_ref[...], 0.0)
    h = jnp.maximum(
        jnp.dot(h, W2_ref[...], preferred_element_type=jnp.float32)
        + b2_ref[...], 0.0)
    deep = jnp.dot(h, Wp_ref[...], preferred_element_type=jnp.float32) \
        + bp_ref[...]

    out_ref[...] = jax.nn.sigmoid(linear + fm + deep)


def _tc_dense(emb, vals, ling, W1, b1, W2, b2, Wp, bp, lb, tb=512):
    B = emb.shape[0]
    fe = F_DIM * E_DIM
    h1, h2 = W1.shape[1], W2.shape[1]
    grid = (B // tb,)
    full = lambda shape: pl.BlockSpec(shape, lambda i: (0, 0))
    return pl.pallas_call(
        _tc_body,
        grid=grid,
        in_specs=[
            pl.BlockSpec((tb, fe), lambda i: (i, 0)),
            pl.BlockSpec((tb, F_DIM), lambda i: (i, 0)),
            pl.BlockSpec((tb, F_DIM), lambda i: (i, 0)),
            full((fe, h1)),
            full((1, h1)),
            full((h1, h2)),
            full((1, h2)),
            full((h2, 1)),
            full((1, 1)),
            full((1, 1)),
        ],
        out_specs=pl.BlockSpec((tb, 1), lambda i: (i, 0)),
        out_shape=jax.ShapeDtypeStruct((B, 1), jnp.float32),
    )(emb, vals, ling, W1, b1, W2, b2, Wp, bp, lb)


def kernel(feature_idx, feature_vals, feature_embedding, linear_w, linear_b,
           W1, b1, W2, b2, Wp, bp):
    B, F = feature_idx.shape
    n_per_w = B * F // NW
    C = n_per_w // CHUNK
    idx_r = feature_idx.reshape(NW, C, CHUNK)
    table_rm = _tc_transpose(feature_embedding.T)
    lin_g = _sc_gather_lin(idx_r, linear_w.reshape(-1))
    emb_g = _sc_gather_emb(idx_r, table_rm)
    emb_flat = emb_g.reshape(B, F * E_DIM)
    lin_flat = lin_g.reshape(B, F)
    return _tc_dense(
        emb_flat, feature_vals, lin_flat,
        W1, b1.reshape(1, -1), W2, b2.reshape(1, -1),
        Wp, bp.reshape(1, 1), linear_b.reshape(1, 1))


# speedup vs baseline: 1.4360x; 1.4360x over previous
import functools

import jax
import jax.numpy as jnp
from jax import lax
from jax.experimental import pallas as pl
from jax.experimental.pallas import tpu as pltpu
from jax.experimental.pallas import tpu_sc as plsc

F_DIM = 26
E_DIM = 16
NC = 2
NS = 16
NW = NC * NS
CHUNK = 128
TBLK = 2048


_SC_MESH = plsc.VectorSubcoreMesh(core_axis_name="c", subcore_axis_name="s")


def _sc_gather_emb(idx_r, table):
    C = idx_r.shape[1]
    n_per_w = C * CHUNK

    @functools.partial(
        pl.kernel,
        out_type=jax.ShapeDtypeStruct((NW, n_per_w, E_DIM), jnp.bfloat16),
        mesh=_SC_MESH,
        scratch_types=[
            pltpu.VMEM((C, CHUNK), jnp.int32),
            pltpu.VMEM((n_per_w, E_DIM), jnp.bfloat16),
            pltpu.SemaphoreType.DMA,
        ],
        compiler_params=pltpu.CompilerParams(use_tc_tiling_on_sc=False),
    )
    def sc_kernel(idx_hbm, table_hbm, emb_out, idx_v, rows_v, sem):
        wid = lax.axis_index("s") * NC + lax.axis_index("c")
        pltpu.sync_copy(idx_hbm.at[wid], idx_v)

        def fire(ci, _):
            pltpu.async_copy(
                table_hbm.at[idx_v.at[ci]],
                rows_v.at[pl.ds(ci * CHUNK, CHUNK)], sem)
            return 0

        lax.fori_loop(0, C, fire, 0)

        def drain(ci, _):
            pltpu.make_async_copy(
                table_hbm.at[idx_v.at[ci]],
                rows_v.at[pl.ds(ci * CHUNK, CHUNK)], sem).wait()
            return 0

        lax.fori_loop(0, C, drain, 0)
        pltpu.sync_copy(rows_v, emb_out.at[wid])

    return sc_kernel(idx_r, table)


def _sc_gather_lin(idx_r, lin_w):
    C = idx_r.shape[1]
    n_per_w = C * CHUNK

    @functools.partial(
        pl.kernel,
        out_type=jax.ShapeDtypeStruct((NW, n_per_w), jnp.float32),
        mesh=_SC_MESH,
        scratch_types=[
            pltpu.VMEM((C, CHUNK), jnp.int32),
            pltpu.VMEM((n_per_w,), jnp.float32),
            pltpu.SemaphoreType.DMA,
        ],
        compiler_params=pltpu.CompilerParams(use_tc_tiling_on_sc=False),
    )
    def sc_kernel(idx_hbm, lin_hbm, lin_out, idx_v, linr_v, sem):
        wid = lax.axis_index("s") * NC + lax.axis_index("c")
        pltpu.sync_copy(idx_hbm.at[wid], idx_v)

        def fire(ci, _):
            pltpu.async_copy(
                lin_hbm.at[idx_v.at[ci]],
                linr_v.at[pl.ds(ci * CHUNK, CHUNK)], sem)
            return 0

        lax.fori_loop(0, C, fire, 0)

        def drain(ci, _):
            pltpu.make_async_copy(
                lin_hbm.at[idx_v.at[ci]],
                linr_v.at[pl.ds(ci * CHUNK, CHUNK)], sem).wait()
            return 0

        lax.fori_loop(0, C, drain, 0)
        pltpu.sync_copy(linr_v, lin_out.at[wid])

    return sc_kernel(idx_r, lin_w)


def _tc_body(emb_ref, vals_ref, ling_ref, W1_ref, b1_ref, W2_ref, b2_ref,
             Wp_ref, bp_ref, lb_ref, out_ref):
    emb = emb_ref[...].astype(jnp.float32)
    vals = vals_ref[...]
    ling = ling_ref[...]

    fe = F_DIM * E_DIM
    jf = lax.broadcasted_iota(jnp.int32, (F_DIM, fe), 1) // E_DIM
    ff = lax.broadcasted_iota(jnp.int32, (F_DIM, fe), 0)
    expand = (jf == ff).astype(jnp.float32)
    w = emb * jnp.dot(vals, expand, preferred_element_type=jnp.float32)

    linear = jnp.sum(ling * vals, axis=1, keepdims=True) + lb_ref[0, 0]

    jj = lax.broadcasted_iota(jnp.int32, (fe, E_DIM), 0)
    dd = lax.broadcasted_iota(jnp.int32, (fe, E_DIM), 1)
    fold = (jj % E_DIM == dd).astype(jnp.float32)
    s = jnp.dot(w, fold, preferred_element_type=jnp.float32)
    fm = 0.5 * (jnp.sum(s * s, axis=1, keepdims=True)
                - jnp.sum(w * w, axis=1, keepdims=True))

    h = jnp.maximum(
        jnp.dot(w, W1_ref[...], preferred_element_type=jnp.float32)
        + b1_ref[...], 0.0)
    h = jnp.maximum(
        jnp.dot(h, W2_ref[...], preferred_element_type=jnp.float32)
        + b2_ref[...], 0.0)
    deep = jnp.dot(h, Wp_ref[...], preferred_element_type=jnp.float32) \
        + bp_ref[...]

    out_ref[...] = jax.nn.sigmoid(linear + fm + deep)


def _tc_dense(emb, vals, ling, W1, b1, W2, b2, Wp, bp, lb, tb=512):
    B = emb.shape[0]
    fe = F_DIM * E_DIM
    h1, h2 = W1.shape[1], W2.shape[1]
    grid = (B // tb,)
    full = lambda shape: pl.BlockSpec(shape, lambda i: (0, 0))
    return pl.pallas_call(
        _tc_body,
        grid=grid,
        in_specs=[
            pl.BlockSpec((tb, fe), lambda i: (i, 0)),
            pl.BlockSpec((tb, F_DIM), lambda i: (i, 0)),
            pl.BlockSpec((tb, F_DIM), lambda i: (i, 0)),
            full((fe, h1)),
            full((1, h1)),
            full((h1, h2)),
            full((1, h2)),
            full((h2, 1)),
            full((1, 1)),
            full((1, 1)),
        ],
        out_specs=pl.BlockSpec((tb, 1), lambda i: (i, 0)),
        out_shape=jax.ShapeDtypeStruct((B, 1), jnp.float32),
    )(emb, vals, ling, W1, b1, W2, b2, Wp, bp, lb)


def kernel(feature_idx, feature_vals, feature_embedding, linear_w, linear_b,
           W1, b1, W2, b2, Wp, bp):
    B, F = feature_idx.shape
    n_per_w = B * F // NW
    C = n_per_w // CHUNK
    idx_r = feature_idx.reshape(NW, C, CHUNK)
    table_bf = feature_embedding.astype(jnp.bfloat16)
    lin_g = _sc_gather_lin(idx_r, linear_w.T.reshape(-1))
    emb_g = _sc_gather_emb(idx_r, table_bf)
    emb_flat = emb_g.reshape(B, F * E_DIM)
    lin_flat = lin_g.reshape(B, F)
    return _tc_dense(
        emb_flat, feature_vals, lin_flat,
        W1, b1.reshape(1, -1), W2, b2.reshape(1, -1),
        Wp, bp.reshape(1, 1), linear_b.reshape(1, 1))


# speedup vs baseline: 1.5725x; 1.0950x over previous
import functools

import jax
import jax.numpy as jnp
from jax import lax
from jax.experimental import pallas as pl
from jax.experimental.pallas import tpu as pltpu
from jax.experimental.pallas import tpu_sc as plsc

F_DIM = 26
E_DIM = 16
NC = 2
NS = 16
NW = NC * NS
CHUNK = 128
TBLK = 2048


_SC_MESH = plsc.VectorSubcoreMesh(core_axis_name="c", subcore_axis_name="s")


ROWS_PER_LINE = 128 // E_DIM


def _sc_gather_emb(idx_r, table128):
    C = idx_r.shape[1]
    n_per_w = C * CHUNK

    out_rows = n_per_w * E_DIM // 128

    @functools.partial(
        pl.kernel,
        out_type=jax.ShapeDtypeStruct((NW, out_rows, 128), jnp.float32),
        mesh=_SC_MESH,
        scratch_types=[
            pltpu.VMEM((C, CHUNK), jnp.int32),
            pltpu.VMEM((C, CHUNK), jnp.int32),
            pltpu.VMEM((2, CHUNK, 128), jnp.float32),
            pltpu.VMEM((out_rows, 128), jnp.float32),
            pltpu.SemaphoreType.DMA,
        ],
        compiler_params=pltpu.CompilerParams(
            use_tc_tiling_on_sc=True, needs_layout_passes=False),
    )
    def sc_kernel(idx_hbm, table_hbm, emb_out, idx_v, idxg_v, gbuf, rows_v,
                  sem):
        wid = lax.axis_index("s") * NC + lax.axis_index("c")
        pltpu.sync_copy(idx_hbm.at[wid], idx_v)

        def xform(k, _):
            ci = k // (CHUNK // 16)
            off = (k % (CHUNK // 16)) * 16
            v = idx_v[ci, pl.ds(off, 16)]
            idxg_v[ci, pl.ds(off, 16)] = lax.shift_right_logical(v, 3)
            return 0

        lax.fori_loop(0, C * (CHUNK // 16), xform, 0)

        def line_copy(ci, buf):
            return pltpu.make_async_copy(
                table_hbm.at[idxg_v.at[ci]], gbuf.at[buf], sem)

        line_copy(0, 0).start()
        ones = jnp.zeros((16,), jnp.int32)
        lanes = lax.iota(jnp.int32, 16)

        def chunk_body(ci, _):
            buf = lax.rem(ci, 2)

            @pl.when(ci + 1 < C)
            def _():
                line_copy(ci + 1, lax.rem(ci + 1, 2)).start()

            line_copy(ci, buf).wait()

            def extract(g, _):
                r = idx_v[ci, pl.ds(g * 16, 16)]
                off = lax.shift_left(lax.bitwise_and(r, 7), 4)
                dst0 = (lanes + ci * CHUNK + g * 16) * E_DIM

                def lane(c, _):
                    vals = plsc.load_gather(
                        gbuf, [ones + buf, lanes + g * 16, off + c])
                    w = dst0 + c
                    plsc.store_scatter(
                        rows_v,
                        [lax.shift_right_logical(w, 7),
                         lax.bitwise_and(w, 127)], vals)
                    return 0

                lax.fori_loop(0, E_DIM, lane, 0)
                return 0

            lax.fori_loop(0, CHUNK // 16, extract, 0)
            return 0

        lax.fori_loop(0, C, chunk_body, 0)
        pltpu.sync_copy(rows_v, emb_out.at[wid])

    return sc_kernel(idx_r, table128)


def _sc_gather_lin(idx_r, lin_w):
    C = idx_r.shape[1]
    n_per_w = C * CHUNK

    @functools.partial(
        pl.kernel,
        out_type=jax.ShapeDtypeStruct((NW, n_per_w), jnp.float32),
        mesh=_SC_MESH,
        scratch_types=[
            pltpu.VMEM((C, CHUNK), jnp.int32),
            pltpu.VMEM((n_per_w,), jnp.float32),
            pltpu.SemaphoreType.DMA,
        ],
        compiler_params=pltpu.CompilerParams(use_tc_tiling_on_sc=False),
    )
    def sc_kernel(idx_hbm, lin_hbm, lin_out, idx_v, linr_v, sem):
        wid = lax.axis_index("s") * NC + lax.axis_index("c")
        pltpu.sync_copy(idx_hbm.at[wid], idx_v)

        def fire(ci, _):
            pltpu.async_copy(
                lin_hbm.at[idx_v.at[ci]],
                linr_v.at[pl.ds(ci * CHUNK, CHUNK)], sem)
            return 0

        lax.fori_loop(0, C, fire, 0)

        def drain(ci, _):
            pltpu.make_async_copy(
                lin_hbm.at[idx_v.at[ci]],
                linr_v.at[pl.ds(ci * CHUNK, CHUNK)], sem).wait()
            return 0

        lax.fori_loop(0, C, drain, 0)
        pltpu.sync_copy(linr_v, lin_out.at[wid])

    return sc_kernel(idx_r, lin_w)


def _tc_body(emb_ref, vals_ref, ling_ref, W1_ref, b1_ref, W2_ref, b2_ref,
             Wp_ref, bp_ref, lb_ref, out_ref):
    emb = emb_ref[...]
    vals = vals_ref[...]
    ling = ling_ref[...]

    fe = F_DIM * E_DIM
    jf = lax.broadcasted_iota(jnp.int32, (F_DIM, fe), 1) // E_DIM
    ff = lax.broadcasted_iota(jnp.int32, (F_DIM, fe), 0)
    expand = (jf == ff).astype(jnp.float32)
    w = emb * jnp.dot(vals, expand, preferred_element_type=jnp.float32)

    linear = jnp.sum(ling * vals, axis=1, keepdims=True) + lb_ref[0, 0]

    jj = lax.broadcasted_iota(jnp.int32, (fe, E_DIM), 0)
    dd = lax.broadcasted_iota(jnp.int32, (fe, E_DIM), 1)
    fold = (jj % E_DIM == dd).astype(jnp.float32)
    s = jnp.dot(w, fold, preferred_element_type=jnp.float32)
    fm = 0.5 * (jnp.sum(s * s, axis=1, keepdims=True)
                - jnp.sum(w * w, axis=1, keepdims=True))

    h = jnp.maximum(
        jnp.dot(w, W1_ref[...], preferred_element_type=jnp.float32)
        + b1_ref[...], 0.0)
    h = jnp.maximum(
        jnp.dot(h, W2_ref[...], preferred_element_type=jnp.float32)
        + b2_ref[...], 0.0)
    deep = jnp.dot(h, Wp_ref[...], preferred_element_type=jnp.float32) \
        + bp_ref[...]

    out_ref[...] = jax.nn.sigmoid(linear + fm + deep)


def _tc_dense(emb, vals, ling, W1, b1, W2, b2, Wp, bp, lb, tb=512):
    B = emb.shape[0]
    fe = F_DIM * E_DIM
    h1, h2 = W1.shape[1], W2.shape[1]
    grid = (B // tb,)
    full = lambda shape: pl.BlockSpec(shape, lambda i: (0, 0))
    return pl.pallas_call(
        _tc_body,
        grid=grid,
        in_specs=[
            pl.BlockSpec((tb, fe), lambda i: (i, 0)),
            pl.BlockSpec((tb, F_DIM), lambda i: (i, 0)),
            pl.BlockSpec((tb, F_DIM), lambda i: (i, 0)),
            full((fe, h1)),
            full((1, h1)),
            full((h1, h2)),
            full((1, h2)),
            full((h2, 1)),
            full((1, 1)),
            full((1, 1)),
        ],
        out_specs=pl.BlockSpec((tb, 1), lambda i: (i, 0)),
        out_shape=jax.ShapeDtypeStruct((B, 1), jnp.float32),
    )(emb, vals, ling, W1, b1, W2, b2, Wp, bp, lb)


def kernel(feature_idx, feature_vals, feature_embedding, linear_w, linear_b,
           W1, b1, W2, b2, Wp, bp):
    B, F = feature_idx.shape
    n_per_w = B * F // NW
    C = n_per_w // CHUNK
    idx_r = feature_idx.reshape(NW, C, CHUNK)
    V = feature_embedding.shape[0]
    table128 = feature_embedding.reshape(V * E_DIM // 128, 128)
    lin_g = _sc_gather_lin(idx_r, linear_w.T.reshape(-1))
    emb_g = _sc_gather_emb(idx_r, table128)
    emb_flat = emb_g.reshape(B, F * E_DIM)
    lin_flat = lin_g.reshape(B, F)
    return _tc_dense(
        emb_flat, feature_vals, lin_flat,
        W1, b1.reshape(1, -1), W2, b2.reshape(1, -1),
        Wp, bp.reshape(1, 1), linear_b.reshape(1, 1))


# speedup vs baseline: 1.6473x; 1.0476x over previous
import functools

import jax
import jax.numpy as jnp
from jax import lax
from jax.experimental import pallas as pl
from jax.experimental.pallas import tpu as pltpu
from jax.experimental.pallas import tpu_sc as plsc

F_DIM = 26
E_DIM = 16
NC = 2
NS = 16
NW = NC * NS
CHUNK = 128

_SC_MESH = plsc.VectorSubcoreMesh(core_axis_name="c", subcore_axis_name="s")


def _sc_gather(idx_r, table, lin_w):
    C = idx_r.shape[1]
    n_per_w = C * CHUNK

    @functools.partial(
        pl.kernel,
        out_type=[
            jax.ShapeDtypeStruct((NW, n_per_w, E_DIM), jnp.float32),
            jax.ShapeDtypeStruct((NW, n_per_w), jnp.float32),
        ],
        mesh=_SC_MESH,
        scratch_types=[
            pltpu.VMEM((C, CHUNK), jnp.int32),
            pltpu.VMEM((n_per_w, E_DIM), jnp.float32),
            pltpu.VMEM((n_per_w,), jnp.float32),
            pltpu.SemaphoreType.DMA,
            pltpu.SemaphoreType.DMA,
        ],
        compiler_params=pltpu.CompilerParams(use_tc_tiling_on_sc=False),
    )
    def sc_kernel(idx_hbm, table_hbm, lin_hbm, emb_out, lin_out,
                  idx_v, rows_v, linr_v, sem_e, sem_l):
        wid = lax.axis_index("s") * NC + lax.axis_index("c")
        pltpu.sync_copy(idx_hbm.at[wid], idx_v)

        def fire(ci, _):
            pltpu.async_copy(
                table_hbm.at[idx_v.at[ci]],
                rows_v.at[pl.ds(ci * CHUNK, CHUNK)], sem_e)
            pltpu.async_copy(
                lin_hbm.at[idx_v.at[ci]],
                linr_v.at[pl.ds(ci * CHUNK, CHUNK)], sem_l)
            return 0

        lax.fori_loop(0, C, fire, 0)

        def drain(ci, _):
            pltpu.make_async_copy(
                table_hbm.at[idx_v.at[ci]],
                rows_v.at[pl.ds(ci * CHUNK, CHUNK)], sem_e).wait()
            pltpu.make_async_copy(
                lin_hbm.at[idx_v.at[ci]],
                linr_v.at[pl.ds(ci * CHUNK, CHUNK)], sem_l).wait()
            return 0

        lax.fori_loop(0, C, drain, 0)
        pltpu.sync_copy(rows_v, emb_out.at[wid])
        pltpu.sync_copy(linr_v, lin_out.at[wid])

    return sc_kernel(idx_r, table, lin_w)


def _tc_body(emb_ref, vals_ref, ling_ref, W1_ref, b1_ref, W2_ref, b2_ref,
             Wp_ref, bp_ref, lb_ref, out_ref):
    emb = emb_ref[...]
    vals = vals_ref[...]
    ling = ling_ref[...]

    fe = F_DIM * E_DIM
    jf = lax.broadcasted_iota(jnp.int32, (F_DIM, fe), 1) // E_DIM
    ff = lax.broadcasted_iota(jnp.int32, (F_DIM, fe), 0)
    expand = (jf == ff).astype(jnp.float32)
    w = emb * jnp.dot(vals, expand, preferred_element_type=jnp.float32)

    linear = jnp.sum(ling * vals, axis=1, keepdims=True) + lb_ref[0, 0]

    jj = lax.broadcasted_iota(jnp.int32, (fe, E_DIM), 0)
    dd = lax.broadcasted_iota(jnp.int32, (fe, E_DIM), 1)
    fold = (jj % E_DIM == dd).astype(jnp.float32)
    s = jnp.dot(w, fold, preferred_element_type=jnp.float32)
    fm = 0.5 * (jnp.sum(s * s, axis=1, keepdims=True)
                - jnp.sum(w * w, axis=1, keepdims=True))

    h = jnp.maximum(
        jnp.dot(w, W1_ref[...], preferred_element_type=jnp.float32)
        + b1_ref[...], 0.0)
    h = jnp.maximum(
        jnp.dot(h, W2_ref[...], preferred_element_type=jnp.float32)
        + b2_ref[...], 0.0)
    deep = jnp.dot(h, Wp_ref[...], preferred_element_type=jnp.float32) \
        + bp_ref[...]

    out_ref[...] = jax.nn.sigmoid(linear + fm + deep)


def _tc_dense(emb, vals, ling, W1, b1, W2, b2, Wp, bp, lb, tb=512):
    B = emb.shape[0]
    fe = F_DIM * E_DIM
    h1, h2 = W1.shape[1], W2.shape[1]
    grid = (B // tb,)
    full = lambda shape: pl.BlockSpec(shape, lambda i: (0, 0))
    return pl.pallas_call(
        _tc_body,
        grid=grid,
        in_specs=[
            pl.BlockSpec((tb, fe), lambda i: (i, 0)),
            pl.BlockSpec((tb, F_DIM), lambda i: (i, 0)),
            pl.BlockSpec((tb, F_DIM), lambda i: (i, 0)),
            full((fe, h1)),
            full((1, h1)),
            full((h1, h2)),
            full((1, h2)),
            full((h2, 1)),
            full((1, 1)),
            full((1, 1)),
        ],
        out_specs=pl.BlockSpec((tb, 1), lambda i: (i, 0)),
        out_shape=jax.ShapeDtypeStruct((B, 1), jnp.float32),
    )(emb, vals, ling, W1, b1, W2, b2, Wp, bp, lb)


def kernel(feature_idx, feature_vals, feature_embedding, linear_w, linear_b,
           W1, b1, W2, b2, Wp, bp):
    B, F = feature_idx.shape
    n_per_w = B * F // NW
    C = n_per_w // CHUNK
    idx_r = feature_idx.reshape(NW, C, CHUNK)
    emb_g, lin_g = _sc_gather(idx_r, feature_embedding,
                              linear_w.T.reshape(-1))
    emb_flat = emb_g.reshape(B, F * E_DIM)
    lin_flat = lin_g.reshape(B, F)
    return _tc_dense(
        emb_flat, feature_vals, lin_flat,
        W1, b1.reshape(1, -1), W2, b2.reshape(1, -1),
        Wp, bp.reshape(1, 1), linear_b.reshape(1, 1))
